# baseline (speedup 1.0000x reference)
import functools

import jax
import jax.numpy as jnp
from jax import lax
from jax.experimental import pallas as pl
from jax.experimental.pallas import tpu as pltpu
from jax.experimental.pallas import tpu_sc as plsc

_B = 1024
_K = 1000
_D = 128
_KP = 1024
_BB = 32
_TOPK = 5

_NW = 32
_NCHUNK = _KP // 16
_SEG = 4
_BSEG = _B // _SEG
_ROWS_PER_W = _BSEG // _NW


def _dist_body(q_ref, f_ref, d_ref):
    f = f_ref[...]
    q = q_ref[...]
    diff = f - q[:, None, :]
    d2 = jnp.sum(diff * diff, axis=-1)
    pad = jnp.full((_BB, _KP - _K), jnp.inf, jnp.float32)
    d_ref[...] = jnp.sqrt(jnp.concatenate([d2, pad], axis=1))


def _distances(queries, features, seg):
    off = seg * (_BSEG // _BB)
    return pl.pallas_call(
        _dist_body,
        grid=(_BSEG // _BB,),
        in_specs=[
            pl.BlockSpec((_BB, _D), lambda i: (i + off, 0)),
            pl.BlockSpec((_BB, _K, _D), lambda i: (i + off, 0, 0)),
        ],
        out_specs=pl.BlockSpec((_BB, _KP), lambda i: (i, 0)),
        out_shape=jax.ShapeDtypeStruct((_BSEG, _KP), jnp.float32),
        compiler_params=pltpu.CompilerParams(
            vmem_limit_bytes=100 * 1024 * 1024),
    )(queries, features)


def _topk_gather_body(seg, dist_hbm, feat_hbm, out_hbm,
                      d0, d1, idx_all, rows_all,
                      sd0, sd1, sg, so):
    c = lax.axis_index("c")
    s = lax.axis_index("s")
    wid = s * 2 + c
    base = wid * _ROWS_PER_W
    lane = lax.broadcasted_iota(jnp.int32, (16,), 0)
    inf_v = jnp.full((16,), jnp.inf, jnp.float32)
    big_i = jnp.full((16,), jnp.int32(2**31 - 1))
    dnums = lax.GatherDimensionNumbers(
        offset_dims=(), collapsed_slice_dims=(0,), start_index_map=(0,))

    def allmin(x):
        for sh in (1, 2, 4, 8):
            perm = lane ^ sh
            shuf = lax.gather(
                x, perm[:, None], dnums, slice_sizes=(1,),
                mode=lax.GatherScatterMode.PROMISE_IN_BOUNDS)
            x = jnp.minimum(x, shuf)
        return x

    def top5(dist_v, off):
        m = [inf_v] * _TOPK
        im = [big_i] * _TOPK
        for j in range(_NCHUNK):
            v = dist_v[pl.ds(off + 16 * j, 16)]
            vi = lane + jnp.int32(16 * j)
            c = [v < m[t] for t in range(_TOPK)]
            nm = [jnp.where(c[0], v, m[0])]
            nim = [jnp.where(c[0], vi, im[0])]
            for t in range(1, _TOPK):
                nm.append(jnp.where(c[t], jnp.maximum(v, m[t - 1]), m[t]))
                nim.append(jnp.where(c[t], jnp.where(c[t - 1], im[t - 1], vi),
                                     im[t]))
            m, im = nm, nim
        sels = []
        for _ in range(_TOPK):
            gm = allmin(m[0])
            sel = allmin(jnp.where(m[0] == gm, im[0], big_i))
            sels.append(sel)
            rm = im[0] == sel
            for t in range(_TOPK - 1):
                m[t] = jnp.where(rm, m[t + 1], m[t])
                im[t] = jnp.where(rm, im[t + 1], im[t])
            m[_TOPK - 1] = jnp.where(rm, inf_v, m[_TOPK - 1])
            im[_TOPK - 1] = jnp.where(rm, big_i, im[_TOPK - 1])
        gv = jnp.where(lane == 0, sels[0], sels[4])
        for t in range(1, _TOPK):
            gv = jnp.where(lane == t, sels[t], gv)
        return gv

    half_rows = _ROWS_PER_W // 2
    pltpu.make_async_copy(
        dist_hbm.at[pl.ds(base * _KP, half_rows * _KP)], d0, sd0).start()
    pltpu.make_async_copy(
        dist_hbm.at[pl.ds((base + half_rows) * _KP, half_rows * _KP)],
        d1, sd1).start()

    for h, (d, sd) in enumerate(((d0, sd0), (d1, sd1))):
        pltpu.make_async_copy(
            dist_hbm.at[pl.ds((base + h * half_rows) * _KP,
                              half_rows * _KP)], d, sd).wait()
        for r in range(half_rows):
            q = h * half_rows + r
            gv = top5(d, r * _KP) \
                + jnp.full((16,), seg * _BSEG + base + q, jnp.int32) \
                * jnp.int32(_K)
            idx_all[pl.ds(16 * q, 16)] = gv

    pltpu.async_copy(feat_hbm.at[idx_all], rows_all, sg).wait()
    for q in range(_ROWS_PER_W):
        pltpu.make_async_copy(rows_all.at[pl.ds(16 * q, _TOPK)],
                              out_hbm.at[base + q], so).start()
    for q in range(_ROWS_PER_W):
        pltpu.make_async_copy(rows_all.at[pl.ds(16 * q, _TOPK)],
                              out_hbm.at[base + q], so).wait()


@functools.cache
def _topk_gather(seg):
    return pl.kernel(
        functools.partial(_topk_gather_body, seg),
        out_type=jax.ShapeDtypeStruct((_BSEG, _TOPK, _D), jnp.float32),
        mesh=plsc.VectorSubcoreMesh(core_axis_name="c", subcore_axis_name="s"),
        scratch_types=[
            pltpu.VMEM((_ROWS_PER_W // 2 * _KP,), jnp.float32),
            pltpu.VMEM((_ROWS_PER_W // 2 * _KP,), jnp.float32),
            pltpu.VMEM((16 * _ROWS_PER_W,), jnp.int32),
            pltpu.VMEM((16 * _ROWS_PER_W, _D), jnp.float32),
            pltpu.SemaphoreType.DMA,
            pltpu.SemaphoreType.DMA,
            pltpu.SemaphoreType.DMA,
            pltpu.SemaphoreType.DMA,
        ],
    )


def kernel(queries, features):
    feat_flat = features.reshape(_B * _K, _D)
    outs = []
    for seg in range(_SEG):
        dist = _distances(queries, features, seg)
        outs.append(_topk_gather(seg)(dist.reshape(-1), feat_flat))
    return jnp.concatenate(outs, axis=0)


# speedup vs baseline: 1.0273x; 1.0273x over previous
import functools

import jax
import jax.numpy as jnp
from jax import lax
from jax.experimental import pallas as pl
from jax.experimental.pallas import tpu as pltpu
from jax.experimental.pallas import tpu_sc as plsc

_B = 1024
_K = 1000
_D = 128
_KP = 1024
_BB = 32
_TOPK = 5

_NW = 32
_NCHUNK = _KP // 16
_SEG = 4
_BSEG = _B // _SEG
_ROWS_PER_W = _BSEG // _NW


def _dist_body(q_ref, f_ref, d_ref):
    f = f_ref[...]
    q = q_ref[...]
    diff = f - q[:, None, :]
    d2 = jnp.sum(diff * diff, axis=-1)
    pad = jnp.full((_BB, _KP - _K), jnp.inf, jnp.float32)
    d_ref[...] = jnp.sqrt(jnp.concatenate([d2, pad], axis=1))


def _distances(queries, features, seg):
    off = seg * (_BSEG // _BB)
    return pl.pallas_call(
        _dist_body,
        grid=(_BSEG // _BB,),
        in_specs=[
            pl.BlockSpec((_BB, _D), lambda i: (i + off, 0)),
            pl.BlockSpec((_BB, _K, _D), lambda i: (i + off, 0, 0)),
        ],
        out_specs=pl.BlockSpec((_BB, _KP), lambda i: (i, 0)),
        out_shape=jax.ShapeDtypeStruct((_BSEG, _KP), jnp.float32),
        compiler_params=pltpu.CompilerParams(
            vmem_limit_bytes=100 * 1024 * 1024),
    )(queries, features)


def _topk_gather_body(seg, dist_hbm, feat_hbm, out_hbm,
                      d0, d1, idx0, idx1, rows0, rows1,
                      sd0, sd1, sg, so0, so1):
    c = lax.axis_index("c")
    s = lax.axis_index("s")
    wid = s * 2 + c
    base = wid * _ROWS_PER_W
    lane = lax.broadcasted_iota(jnp.int32, (16,), 0)
    inf_v = jnp.full((16,), jnp.inf, jnp.float32)
    big_i = jnp.full((16,), jnp.int32(2**31 - 1))
    dnums = lax.GatherDimensionNumbers(
        offset_dims=(), collapsed_slice_dims=(0,), start_index_map=(0,))

    def dist_src(row):
        return dist_hbm.at[pl.ds(row * _KP, _KP)]

    def allmin(x):
        for sh in (1, 2, 4, 8):
            perm = lane ^ sh
            shuf = lax.gather(
                x, perm[:, None], dnums, slice_sizes=(1,),
                mode=lax.GatherScatterMode.PROMISE_IN_BOUNDS)
            x = jnp.minimum(x, shuf)
        return x

    def top5(dist_v):
        m = [inf_v] * _TOPK
        im = [big_i] * _TOPK
        for j in range(_NCHUNK):
            v = dist_v[pl.ds(16 * j, 16)]
            vi = lane + jnp.int32(16 * j)
            for t in range(_TOPK):
                lt = v < m[t]
                m_new = jnp.where(lt, v, m[t])
                i_new = jnp.where(lt, vi, im[t])
                v, vi = jnp.where(lt, m[t], v), jnp.where(lt, im[t], vi)
                m[t], im[t] = m_new, i_new
        sels = []
        for _ in range(_TOPK):
            gm = allmin(m[0])
            sel = allmin(jnp.where(m[0] == gm, im[0], big_i))
            sels.append(sel)
            rm = im[0] == sel
            for t in range(_TOPK - 1):
                m[t] = jnp.where(rm, m[t + 1], m[t])
                im[t] = jnp.where(rm, im[t + 1], im[t])
            m[_TOPK - 1] = jnp.where(rm, inf_v, m[_TOPK - 1])
            im[_TOPK - 1] = jnp.where(rm, big_i, im[_TOPK - 1])
        gv = jnp.where(lane == 0, sels[0], sels[4])
        for t in range(1, _TOPK):
            gv = jnp.where(lane == t, sels[t], gv)
        return gv

    pltpu.make_async_copy(dist_src(base), d0, sd0).start()
    pltpu.make_async_copy(dist_src(base + 1), d1, sd1).start()

    def half(p, row, d, idx_v, rows_v, sd, so):
        pltpu.make_async_copy(dist_src(row), d, sd).wait()
        gv = top5(d) + jnp.full((16,), seg * _BSEG + row, jnp.int32) \
            * jnp.int32(_K)

        @pl.when(p > 0)
        def _():
            pltpu.make_async_copy(rows_v.at[pl.ds(0, _TOPK)],
                                  out_hbm.at[row], so).wait()

        idx_v[...] = gv
        pltpu.async_copy(feat_hbm.at[idx_v.at[pl.ds(0, 8)]], rows_v, sg).wait()
        pltpu.make_async_copy(rows_v.at[pl.ds(0, _TOPK)],
                              out_hbm.at[row], so).start()

        @pl.when(row + 2 < base + _ROWS_PER_W)
        def _():
            pltpu.make_async_copy(dist_src(row + 2), d, sd).start()

    def pair(p, carry):
        a = base + 2 * p
        half(p, a, d0, idx0, rows0, sd0, so0)
        half(p, a + 1, d1, idx1, rows1, sd1, so1)
        return carry

    lax.fori_loop(0, _ROWS_PER_W // 2, pair, 0)
    pltpu.make_async_copy(rows0.at[pl.ds(0, _TOPK)],
                          out_hbm.at[base + _ROWS_PER_W - 2], so0).wait()
    pltpu.make_async_copy(rows1.at[pl.ds(0, _TOPK)],
                          out_hbm.at[base + _ROWS_PER_W - 1], so1).wait()


@functools.cache
def _topk_gather(seg):
    return pl.kernel(
        functools.partial(_topk_gather_body, seg),
        out_type=jax.ShapeDtypeStruct((_BSEG, _TOPK, _D), jnp.float32),
        mesh=plsc.VectorSubcoreMesh(core_axis_name="c", subcore_axis_name="s"),
        scratch_types=[
            pltpu.VMEM((_KP,), jnp.float32),
            pltpu.VMEM((_KP,), jnp.float32),
            pltpu.VMEM((16,), jnp.int32),
            pltpu.VMEM((16,), jnp.int32),
            pltpu.VMEM((8, _D), jnp.float32),
            pltpu.VMEM((8, _D), jnp.float32),
            pltpu.SemaphoreType.DMA,
            pltpu.SemaphoreType.DMA,
            pltpu.SemaphoreType.DMA,
            pltpu.SemaphoreType.DMA,
            pltpu.SemaphoreType.DMA,
        ],
    )


def kernel(queries, features):
    feat_flat = features.reshape(_B * _K, _D)
    outs = []
    for seg in range(_SEG):
        dist = _distances(queries, features, seg)
        outs.append(_topk_gather(seg)(dist.reshape(-1), feat_flat))
    return jnp.concatenate(outs, axis=0)
